# Initial kernel scaffold; baseline (speedup 1.0000x reference)
import jax
import jax.numpy as jnp
from jax.experimental import pallas as pl


def kernel(x, pp_edge_index, edge_weight, W_embed, W1, b1):
    raise NotImplementedError("write your pallas kernel here")


# speedup vs baseline: 18.5056x; 18.5056x over previous
import functools

import jax
import jax.numpy as jnp
from jax import lax
from jax.experimental import pallas as pl
from jax.experimental.pallas import tpu as pltpu
from jax.experimental.pallas import tpu_sc as plsc

N = 10000
D = 128
E = 320000

NC = 2
NS = 16
NW = NC * NS
K = 128
C = 79
EPAD = NW * C * K
NPAD = 10240
RPT = NPAD // NS

_mesh = plsc.VectorSubcoreMesh(
    core_axis_name="c", subcore_axis_name="s", num_cores=NC, num_subcores=NS
)


@functools.partial(
    pl.kernel,
    out_type=jax.ShapeDtypeStruct((NC, NPAD), jnp.float32),
    mesh=_mesh,
    scratch_types=[
        pltpu.VMEM((C, K), jnp.int32),
        pltpu.VMEM((C, K), jnp.float32),
        pltpu.VMEM((RPT,), jnp.float32),
        pltpu.VMEM_SHARED((NPAD,), jnp.float32),
    ],
)
def _sc_deg(dst_hbm, w_hbm, out_hbm, dst_v, w_v, buf_v, deg_s):
    cid = lax.axis_index("c")
    sid = lax.axis_index("s")
    wid = sid * NC + cid

    def _zero(i, _):
        buf_v[pl.ds(i * 16, 16)] = jnp.zeros((16,), jnp.float32)
        return 0

    lax.fori_loop(0, RPT // 16, _zero, 0)
    pltpu.sync_copy(buf_v, deg_s.at[pl.ds(sid * RPT, RPT)])
    plsc.subcore_barrier()

    pltpu.sync_copy(dst_hbm.at[wid], dst_v)
    pltpu.sync_copy(w_hbm.at[wid], w_v)

    def _body(j, _):
        pltpu.sync_copy(w_v.at[j], deg_s.at[dst_v.at[j]], add=True)
        return 0

    lax.fori_loop(0, C, _body, 0)
    plsc.subcore_barrier()

    pltpu.sync_copy(deg_s.at[pl.ds(sid * RPT, RPT)], buf_v)
    pltpu.sync_copy(buf_v, out_hbm.at[cid, pl.ds(sid * RPT, RPT)])


@functools.partial(
    pl.kernel,
    out_type=jax.ShapeDtypeStruct((NC, NPAD, D), jnp.float32),
    mesh=_mesh,
    scratch_types=[
        pltpu.VMEM((C, K), jnp.int32),
        pltpu.VMEM((C, K), jnp.int32),
        pltpu.VMEM((C, K), jnp.float32),
        pltpu.VMEM((K, D), jnp.float32),
        pltpu.VMEM_SHARED((NPAD, D), jnp.float32),
        pltpu.SemaphoreType.DMA,
    ],
)
def _sc_spmm(g_hbm, src_hbm, dst_hbm, w_hbm, out_hbm,
             src_v, dst_v, w_v, rows_v, acc_s, sem):
    cid = lax.axis_index("c")
    sid = lax.axis_index("s")
    wid = sid * NC + cid

    def _zero(r, _):
        for i in range(D // 16):
            rows_v[r, pl.ds(i * 16, 16)] = jnp.zeros((16,), jnp.float32)
        return 0

    lax.fori_loop(0, K, _zero, 0)
    for t in range(RPT // K):
        pltpu.sync_copy(rows_v, acc_s.at[pl.ds(sid * RPT + t * K, K)])
    plsc.subcore_barrier()

    pltpu.sync_copy(src_hbm.at[wid], src_v)
    pltpu.sync_copy(dst_hbm.at[wid], dst_v)
    pltpu.sync_copy(w_hbm.at[wid], w_v)

    def _chunk(j, _):
        pltpu.async_copy(g_hbm.at[src_v.at[j]], rows_v, sem).wait()

        def _scale(g, _):
            wvec = w_v[j, pl.ds(g * 16, 16)]
            for r in range(16):
                s = wvec[r]
                for i in range(D // 16):
                    sl = pl.ds(i * 16, 16)
                    rows_v[g * 16 + r, sl] = rows_v[g * 16 + r, sl] * s
            return 0

        lax.fori_loop(0, K // 16, _scale, 0)
        pltpu.sync_copy(rows_v, acc_s.at[dst_v.at[j]], add=True)
        return 0

    lax.fori_loop(0, C, _chunk, 0)
    plsc.subcore_barrier()

    for t in range(RPT // K):
        pltpu.sync_copy(acc_s.at[pl.ds(sid * RPT + t * K, K)], rows_v)
        pltpu.sync_copy(rows_v, out_hbm.at[cid, pl.ds(sid * RPT + t * K, K)])


def _tc_dinv_body(deg_ref, out_ref):
    gid = lax.broadcasted_iota(jnp.int32, (NPAD,), 0)
    deg = deg_ref[0] + deg_ref[1] + jnp.where(gid < N, 1.0, 0.0)
    out_ref[...] = jnp.where(deg > 0, lax.rsqrt(jnp.maximum(deg, 1e-12)), 0.0)


def _tc_embed_body(x_ref, we_ref, w1_ref, dinv_ref, g_ref):
    h0 = jnp.dot(x_ref[...], we_ref[...], preferred_element_type=jnp.float32)
    h1 = jnp.dot(h0, w1_ref[...], preferred_element_type=jnp.float32)
    g_ref[...] = dinv_ref[...] * h1


def _tc_mid_body(p_ref, g_ref, dinv_ref, b_ref, w1_ref, g2_ref):
    s = p_ref[0, :N, :] + p_ref[1, :N, :] + g_ref[...]
    a = jnp.maximum(dinv_ref[...] * s + b_ref[...], 0.0)
    h = jnp.dot(a, w1_ref[...], preferred_element_type=jnp.float32)
    g2_ref[...] = dinv_ref[...] * h


def _tc_final_body(p_ref, g_ref, dinv_ref, b_ref, out_ref):
    s = p_ref[0, :N, :] + p_ref[1, :N, :] + g_ref[...]
    out_ref[...] = dinv_ref[...] * s + b_ref[...]


def _tc_call(body, out_shape, *args):
    return pl.pallas_call(
        body, out_shape=jax.ShapeDtypeStruct(out_shape, jnp.float32)
    )(*args)


def kernel(x, pp_edge_index, edge_weight, W_embed, W1, b1):
    src = pp_edge_index[0]
    dst = pp_edge_index[1]

    pad = EPAD - E
    spread = (jnp.arange(pad, dtype=jnp.int32) * 97) % N
    src_p = jnp.concatenate([src, spread]).reshape(NW, C, K)
    dst_p = jnp.concatenate([dst, spread]).reshape(NW, C, K)
    w_p = jnp.concatenate(
        [edge_weight, jnp.zeros((pad,), jnp.float32)]
    ).reshape(NW, C, K)

    deg_part = _sc_deg(dst_p, w_p)
    dinv = _tc_call(_tc_dinv_body, (NPAD,), deg_part)
    dinv_col = dinv[:N, None]

    b_row = b1[None, :]
    g1 = _tc_call(_tc_embed_body, (N, D), x, W_embed, W1, dinv_col)
    g1_pad = jnp.concatenate([g1, jnp.zeros((NPAD - N, D), jnp.float32)])

    p1 = _sc_spmm(g1_pad, src_p, dst_p, w_p)
    g2 = _tc_call(_tc_mid_body, (N, D), p1, g1, dinv_col, b_row, W1)
    g2_pad = jnp.concatenate([g2, jnp.zeros((NPAD - N, D), jnp.float32)])

    p2 = _sc_spmm(g2_pad, src_p, dst_p, w_p)
    out = _tc_call(_tc_final_body, (N, D), p2, g2, dinv_col, b_row)
    return out
